# baseline (speedup 1.0000x reference)
import functools

import jax
import jax.numpy as jnp
from jax.experimental import pallas as pl
from jax.experimental.pallas import tpu as pltpu
from jax.experimental.pallas import tpu_sc as plsc

INTERPRET = False

DIMS = 2
K = 16
PAD = 128
KNN_T = 512
LA_T = 256
FM_T = 512

_BIG = 3.0e8


def _knn_kernel(pc_ref, pct_ref, idx_ref, *, n):
    b = pl.program_id(0)
    pc = pc_ref[0]
    pct = pct_ref[0]
    cx = pc[:, 0:1]
    cy = pc[:, 1:2]
    jx = pct[0:1, :]
    jy = pct[1:2, :]
    sq_i = cx * cx + cy * cy
    sq_j = jx * jx + jy * jy
    cxb = cx.astype(jnp.bfloat16).astype(jnp.float32)
    cyb = cy.astype(jnp.bfloat16).astype(jnp.float32)
    jxb = jx.astype(jnp.bfloat16).astype(jnp.float32)
    jyb = jy.astype(jnp.bfloat16).astype(jnp.float32)
    cross = cxb * jxb + cyb * jyb
    d2 = (sq_i + sq_j) - 2.0 * cross
    iota = jax.lax.broadcasted_iota(jnp.int32, d2.shape, 1)
    base = b * n
    big_i = jnp.int32(2 ** 30)
    for k in range(K):
        m = jnp.min(d2, axis=1, keepdims=True)
        sel = jnp.where(d2 == m, iota, big_i)
        j = jnp.min(sel, axis=1, keepdims=True)
        idx_ref[0, :, k] = j[:, 0] + base
        d2 = jnp.where(iota == j, jnp.inf, d2)


def _knn(pc, pct):
    b_dim, n, _ = pc.shape
    grid = (b_dim, n // KNN_T)
    return pl.pallas_call(
        functools.partial(_knn_kernel, n=n),
        grid=grid,
        in_specs=[
            pl.BlockSpec((1, KNN_T, 2), lambda b, i: (b, i, 0)),
            pl.BlockSpec((1, 2, n), lambda b, i: (b, 0, 0)),
        ],
        out_specs=pl.BlockSpec((1, KNN_T, K), lambda b, i: (b, i, 0)),
        out_shape=jax.ShapeDtypeStruct((b_dim, n, K), jnp.int32),
        interpret=INTERPRET,
    )(pc, pct)


_GW = 128


def _sc_gather128(table, idx_flat):
    m = idx_flat.shape[0]
    ind2 = idx_flat.reshape(1, m)
    mesh = plsc.VectorSubcoreMesh(core_axis_name="core",
                                  subcore_axis_name="subcore")

    @functools.partial(
        pl.kernel,
        out_type=jax.ShapeDtypeStruct((m, PAD), table.dtype),
        mesh=mesh)
    def kern(x_hbm, i_hbm, o_hbm):
        def body(i_vmem, o_vmem):
            pltpu.sync_copy(x_hbm.at[i_vmem.at[0]], o_vmem)
        pltpu.emit_pipeline(
            body,
            grid=(m // _GW,),
            in_specs=[pl.BlockSpec((1, _GW), index_map=lambda i: (0, i))],
            out_specs=[pl.BlockSpec((_GW, PAD), index_map=lambda i: (i, 0))],
            core_axis_name=('core', 'subcore'),
            dimension_semantics=(pltpu.PARALLEL,),
        )(i_hbm, o_hbm)

    return kern(table, ind2)


def _fm_kernel(f_ref, pc_ref, w_ref, b_ref, p_ref, out_ref):
    out_ref[...] = jnp.maximum(
        jnp.dot(f_ref[...], w_ref[...], preferred_element_type=jnp.float32)
        + b_ref[...], 0.0) + jnp.dot(
            pc_ref[...], p_ref[...], preferred_element_type=jnp.float32)


def _fm(f2d, pc2d, w_pad, b_pad, place):
    rows, iu = f2d.shape
    return pl.pallas_call(
        _fm_kernel,
        grid=(rows // FM_T,),
        in_specs=[
            pl.BlockSpec((FM_T, iu), lambda i: (i, 0)),
            pl.BlockSpec((FM_T, 2), lambda i: (i, 0)),
            pl.BlockSpec((iu, PAD), lambda i: (0, 0)),
            pl.BlockSpec((1, PAD), lambda i: (0, 0)),
            pl.BlockSpec((2, PAD), lambda i: (0, 0)),
        ],
        out_specs=pl.BlockSpec((FM_T, PAD), lambda i: (i, 0)),
        out_shape=jax.ShapeDtypeStruct((rows, PAD), jnp.float32),
        interpret=INTERPRET,
    )(f2d, pc2d, w_pad, b_pad, place)


def _la_kernel(pc_ref, nf_ref, wsp_ref, bsp_ref, watt_ref,
               wp_ref, bp_ref, p_ref, out_ref, *, h, od):
    def epilogue(f2):
        return f2 + jnp.dot(pc_ref[0], p_ref[...],
                            preferred_element_type=jnp.float32)
    _la_core(pc_ref, nf_ref, wsp_ref, bsp_ref, watt_ref,
             wp_ref, bp_ref, out_ref, h=h, od=od, epilogue=epilogue)


def _la2_kernel(pc_ref, nf_ref, wsp_ref, bsp_ref, watt_ref,
                wp_ref, bp_ref, fin_ref, wm2_ref, bm2_ref, wsc_ref,
                bsc_ref, out_ref, *, h, od):
    def epilogue(f2):
        res = jnp.dot(f2, wm2_ref[...], preferred_element_type=jnp.float32)
        sc = jnp.dot(fin_ref[0], wsc_ref[...],
                     preferred_element_type=jnp.float32)
        return jnp.maximum(res + bm2_ref[...] + sc + bsc_ref[...], 0.0)
    _la_core(pc_ref, nf_ref, wsp_ref, bsp_ref, watt_ref,
             wp_ref, bp_ref, out_ref, h=h, od=od, epilogue=epilogue)


def _la_core(pc_ref, nf_ref, wsp_ref, bsp_ref, watt_ref,
             wp_ref, bp_ref, out_ref, *, h, od, epilogue):
    u = 2 * h
    t = LA_T
    pc = pc_ref[0]
    npc = nf_ref[0]
    cx3 = pc[:, 0:1].reshape(t, 1, 1)
    cy3 = pc[:, 1:2].reshape(t, 1, 1)
    nx3 = npc[:, h:h + 1].reshape(t, K, 1)
    ny3 = npc[:, h + 1:h + 2].reshape(t, K, 1)
    rx3 = cx3 - nx3
    ry3 = cy3 - ny3
    dist3 = jnp.sqrt(rx3 * rx3 + ry3 * ry3 + 1e-12)

    wsp = wsp_ref[...]
    wc_x = (wsp[0:1, :] + wsp[4:5, :]).reshape(1, 1, h)
    wc_y = (wsp[1:2, :] + wsp[5:6, :]).reshape(1, 1, h)
    wn_x = (wsp[2:3, :] - wsp[4:5, :]).reshape(1, 1, h)
    wn_y = (wsp[3:4, :] - wsp[5:6, :]).reshape(1, 1, h)
    wd = wsp[6:7, :].reshape(1, 1, h)
    bsp3 = bsp_ref[...].reshape(1, 1, h)
    enc3 = jnp.maximum(
        cx3 * wc_x + cy3 * wc_y + nx3 * wn_x + ny3 * wn_y + dist3 * wd
        + bsp3, 0.0)
    nf3 = nf_ref[0][:, :h].reshape(t, K, h)

    watt = watt_ref[...]
    enc2 = enc3.reshape(t * K, h)
    nf2 = nf3.reshape(t * K, h)
    logits = (jnp.dot(enc2, watt[:h, :], preferred_element_type=jnp.float32)
              + jnp.dot(nf2, watt[h:, :], preferred_element_type=jnp.float32))
    l3 = logits.reshape(t, K, u)
    mx = jnp.max(l3, axis=1, keepdims=True)
    e = jnp.exp(l3 - mx)
    s = jnp.sum(e, axis=1, keepdims=True)
    scores = e / s
    agg_e = jnp.sum(scores[:, :, :h] * enc3, axis=1)
    agg_n = jnp.sum(scores[:, :, h:] * nf3, axis=1)
    wp = wp_ref[...]
    out = (jnp.dot(agg_e, wp[:h, :], preferred_element_type=jnp.float32)
           + jnp.dot(agg_n, wp[h:, :], preferred_element_type=jnp.float32)
           + bp_ref[...])
    f2 = jnp.maximum(out, 0.0)
    if epilogue is not None:
        f2 = epilogue(f2)
    out_ref[0] = f2


def _la_call(pc, nf, wsp8, bsp, watt, wp_pad, bp_pad, od,
             extras=None, extras2=None):
    b_dim, n, _ = pc.shape
    h = watt.shape[0] // 2
    rows = LA_T * K
    grid = (b_dim, n // LA_T)
    in_specs = [
        pl.BlockSpec((1, LA_T, 2), lambda b, i: (b, i, 0)),
        pl.BlockSpec((1, rows, PAD), lambda b, i: (b, i, 0)),
        pl.BlockSpec((8, h), lambda b, i: (0, 0)),
        pl.BlockSpec((1, h), lambda b, i: (0, 0)),
        pl.BlockSpec((2 * h, 2 * h), lambda b, i: (0, 0)),
        pl.BlockSpec((2 * h, od), lambda b, i: (0, 0)),
        pl.BlockSpec((1, od), lambda b, i: (0, 0)),
    ]
    args = [pc, nf, wsp8, bsp, watt, wp_pad, bp_pad]
    if extras is None:
        place = extras2
        in_specs.append(pl.BlockSpec((2, od), lambda b, i: (0, 0)))
        args.append(place)
        kfn = functools.partial(_la_kernel, h=h, od=od)
    else:
        fin, wm2, bm2, wsc, bsc = extras
        iu = wsc.shape[0]
        u = wm2.shape[0]
        in_specs += [
            pl.BlockSpec((1, LA_T, iu), lambda b, i: (b, i, 0)),
            pl.BlockSpec((u, u), lambda b, i: (0, 0)),
            pl.BlockSpec((1, u), lambda b, i: (0, 0)),
            pl.BlockSpec((iu, u), lambda b, i: (0, 0)),
            pl.BlockSpec((1, u), lambda b, i: (0, 0)),
        ]
        args += [fin, wm2, bm2, wsc, bsc]
        kfn = functools.partial(_la2_kernel, h=h, od=od)
    return pl.pallas_call(
        kfn,
        grid=grid,
        in_specs=in_specs,
        out_specs=pl.BlockSpec((1, LA_T, od), lambda b, i: (b, i, 0)),
        out_shape=jax.ShapeDtypeStruct((b_dim, n, od), jnp.float32),
        interpret=INTERPRET,
    )(*args)


def _pad_w(w, width=PAD):
    out_w = w.shape[1]
    if out_w == width:
        return w
    return jnp.concatenate(
        [w, jnp.zeros((w.shape[0], width - out_w), w.dtype)], axis=1)


def _pad_b(b, width=PAD):
    if b.shape[0] == width:
        return b[None, :]
    return jnp.concatenate(
        [b, jnp.zeros((width - b.shape[0],), b.dtype)])[None, :]


def _pad8_rows(w):
    return jnp.concatenate([w, jnp.zeros((1, w.shape[1]), w.dtype)], axis=0)


def kernel(pc, feats, fc_kernel, fc_bias, drb_params):
    b_dim, n, _ = pc.shape

    f = jax.lax.conv_general_dilated(
        feats, fc_kernel, window_strides=(1,), padding='SAME',
        dimension_numbers=('NWC', 'WIO', 'NWC'))
    f = jax.nn.relu(f + fc_bias)

    pct = jnp.transpose(pc, (0, 2, 1))
    idx_g = _knn(pc, pct)
    idx_flat = idx_g.reshape(-1)
    pc2d = pc.reshape(b_dim * n, DIMS)

    for p in drb_params:
        f_in = f
        iu = p['Wm1'].shape[0]
        h = p['Wm1'].shape[1]
        u = 2 * h
        place = jnp.zeros((2, PAD), jnp.float32)
        place = place.at[0, h].set(1.0).at[1, h + 1].set(1.0)
        fm128 = _fm(f_in.reshape(b_dim * n, iu), pc2d,
                    _pad_w(p['Wm1']), _pad_b(p['bm1']), place)
        nf = _sc_gather128(fm128, idx_flat).reshape(b_dim, n * K, PAD)
        f1 = _la_call(pc, nf, _pad8_rows(p['Wsp0']), p['bsp0'][None, :],
                      p['Watt0'], _pad_w(p['Wp0']), _pad_b(p['bp0']), PAD,
                      extras2=place)
        nf2 = _sc_gather128(f1.reshape(b_dim * n, PAD),
                            idx_flat).reshape(b_dim, n * K, PAD)
        f = _la_call(pc, nf2, _pad8_rows(p['Wsp1']), p['bsp1'][None, :],
                     p['Watt1'], p['Wp1'], p['bp1'][None, :], u,
                     extras=(f_in, p['Wm2'], p['bm2'][None, :],
                             p['Wsc'], p['bsc'][None, :]))
    return f


# speedup vs baseline: 1.5447x; 1.5447x over previous
import functools

import jax
import jax.numpy as jnp
from jax.experimental import pallas as pl
from jax.experimental.pallas import tpu as pltpu
from jax.experimental.pallas import tpu_sc as plsc

INTERPRET = False

DIMS = 2
K = 16
PAD = 128
KNN_T = 512
LA_T = 256
FM_T = 512

_BIG = 3.0e8


def _knn_kernel(pc_ref, pct_ref, idx_ref, *, n):
    b = pl.program_id(0)
    pc = pc_ref[0]
    pct = pct_ref[0]
    cx = pc[:, 0:1]
    cy = pc[:, 1:2]
    jx = pct[0:1, :]
    jy = pct[1:2, :]
    sq_i = cx * cx + cy * cy
    sq_j = jx * jx + jy * jy
    cxb = cx.astype(jnp.bfloat16).astype(jnp.float32)
    cyb = cy.astype(jnp.bfloat16).astype(jnp.float32)
    jxb = jx.astype(jnp.bfloat16).astype(jnp.float32)
    jyb = jy.astype(jnp.bfloat16).astype(jnp.float32)
    cross = cxb * jxb + cyb * jyb
    d2 = (sq_i + sq_j) - 2.0 * cross
    iota = jax.lax.broadcasted_iota(jnp.int32, d2.shape, 1)
    base = b * n
    big_i = jnp.int32(2 ** 30)
    for k in range(K):
        m = jnp.min(d2, axis=1, keepdims=True)
        sel = jnp.where(d2 == m, iota, big_i)
        j = jnp.min(sel, axis=1, keepdims=True)
        idx_ref[0, :, k] = j[:, 0] + base
        d2 = jnp.where(iota == j, jnp.inf, d2)


def _knn(pc, pct):
    b_dim, n, _ = pc.shape
    grid = (b_dim, n // KNN_T)
    return pl.pallas_call(
        functools.partial(_knn_kernel, n=n),
        grid=grid,
        in_specs=[
            pl.BlockSpec((1, KNN_T, 2), lambda b, i: (b, i, 0)),
            pl.BlockSpec((1, 2, n), lambda b, i: (b, 0, 0)),
        ],
        out_specs=pl.BlockSpec((1, KNN_T, K), lambda b, i: (b, i, 0)),
        out_shape=jax.ShapeDtypeStruct((b_dim, n, K), jnp.int32),
        interpret=INTERPRET,
    )(pc, pct)


_GW = 128


def _sc_gather128(table, idx_flat):
    m = idx_flat.shape[0]
    ind2 = idx_flat.reshape(1, m)
    mesh = plsc.VectorSubcoreMesh(core_axis_name="core",
                                  subcore_axis_name="subcore")

    @functools.partial(
        pl.kernel,
        out_type=jax.ShapeDtypeStruct((m, PAD), table.dtype),
        mesh=mesh)
    def kern(x_hbm, i_hbm, o_hbm):
        def body(i_vmem, o_vmem):
            pltpu.sync_copy(x_hbm.at[i_vmem.at[0]], o_vmem)
        pltpu.emit_pipeline(
            body,
            grid=(m // _GW,),
            in_specs=[pl.BlockSpec((1, _GW), index_map=lambda i: (0, i))],
            out_specs=[pl.BlockSpec((_GW, PAD), index_map=lambda i: (i, 0))],
            core_axis_name='subcore',
            dimension_semantics=(pltpu.PARALLEL,),
        )(i_hbm, o_hbm)

    return kern(table, ind2)


def _conv_kernel(fp_ref, w_ref, b_ref, out_ref, *, n, taps):
    fp = fp_ref[0]
    acc = jnp.zeros((n, out_ref.shape[-1]), jnp.float32)
    for j in range(taps):
        acc = acc + jnp.dot(fp[j:j + n, :], w_ref[2 * j:2 * j + 2, :],
                            preferred_element_type=jnp.float32)
    out_ref[0] = jnp.maximum(acc + b_ref[...], 0.0)


def _conv(feats, fc_kernel, fc_bias):
    b_dim, n, c = feats.shape
    taps = fc_kernel.shape[0]
    fpad = jnp.pad(feats, ((0, 0), (taps // 2 - 1, taps // 2 + 1), (0, 0)))
    w2d = fc_kernel.reshape(taps * c, fc_kernel.shape[2])
    oc = fc_kernel.shape[2]
    return pl.pallas_call(
        functools.partial(_conv_kernel, n=n, taps=taps),
        grid=(b_dim,),
        in_specs=[
            pl.BlockSpec((1, n + taps, c), lambda b: (b, 0, 0)),
            pl.BlockSpec((taps * c, oc), lambda b: (0, 0)),
            pl.BlockSpec((1, oc), lambda b: (0, 0)),
        ],
        out_specs=pl.BlockSpec((1, n, oc), lambda b: (b, 0, 0)),
        out_shape=jax.ShapeDtypeStruct((b_dim, n, oc), jnp.float32),
        interpret=INTERPRET,
    )(fpad, w2d, fc_bias[None, :])


def _fm_kernel(f_ref, w_ref, b_ref, out_ref):
    out_ref[...] = jnp.maximum(
        jnp.dot(f_ref[...], w_ref[...], preferred_element_type=jnp.float32)
        + b_ref[...], 0.0).astype(out_ref.dtype)


def _fm(f2d, w_pad, b_pad, pack):
    rows, iu = f2d.shape
    w = PAD // pack
    nsteps = rows // FM_T
    per = nsteps // pack
    out_spec = pl.BlockSpec(
        (FM_T, w), lambda i: (i % per, i // per))
    return pl.pallas_call(
        _fm_kernel,
        grid=(nsteps,),
        in_specs=[
            pl.BlockSpec((FM_T, iu), lambda i: (i, 0)),
            pl.BlockSpec((iu, w), lambda i: (0, 0)),
            pl.BlockSpec((1, w), lambda i: (0, 0)),
        ],
        out_specs=out_spec,
        out_shape=jax.ShapeDtypeStruct((rows // pack, PAD), jnp.float32),
        interpret=INTERPRET,
    )(f2d, w_pad, b_pad)


def _la_kernel(pc_ref, npc_ref, nf_ref, wsp_ref, bsp_ref, watt_ref,
               wp_ref, bp_ref, out_ref, *, h, od):
    _la_core(pc_ref, npc_ref, nf_ref, wsp_ref, bsp_ref, watt_ref,
             wp_ref, bp_ref, out_ref, h=h, od=od, epilogue=None)


def _la2_kernel(pc_ref, npc_ref, nf_ref, wsp_ref, bsp_ref, watt_ref,
                wp_ref, bp_ref, fin_ref, wm2_ref, bm2_ref, wsc_ref,
                bsc_ref, out_ref, *, h, od):
    def epilogue(f2):
        res = jnp.dot(f2, wm2_ref[...], preferred_element_type=jnp.float32)
        sc = jnp.dot(fin_ref[0], wsc_ref[...],
                     preferred_element_type=jnp.float32)
        return jnp.maximum(res + bm2_ref[...] + sc + bsc_ref[...], 0.0)
    _la_core(pc_ref, npc_ref, nf_ref, wsp_ref, bsp_ref, watt_ref,
             wp_ref, bp_ref, out_ref, h=h, od=od, epilogue=epilogue)


def _la_core(pc_ref, npc_ref, nf_ref, wsp_ref, bsp_ref, watt_ref,
             wp_ref, bp_ref, out_ref, *, h, od, epilogue):
    u = 2 * h
    t = LA_T
    pc = pc_ref[0]
    npc = npc_ref[0]
    cx3 = pc[:, 0:1].reshape(t, 1, 1)
    cy3 = pc[:, 1:2].reshape(t, 1, 1)
    nx3 = npc[:, 0:1].reshape(t, K, 1)
    ny3 = npc[:, 1:2].reshape(t, K, 1)
    rx3 = cx3 - nx3
    ry3 = cy3 - ny3
    dist3 = jnp.sqrt(rx3 * rx3 + ry3 * ry3 + 1e-12)

    wsp = wsp_ref[...]
    wc_x = (wsp[0:1, :] + wsp[4:5, :]).reshape(1, 1, h)
    wc_y = (wsp[1:2, :] + wsp[5:6, :]).reshape(1, 1, h)
    wn_x = (wsp[2:3, :] - wsp[4:5, :]).reshape(1, 1, h)
    wn_y = (wsp[3:4, :] - wsp[5:6, :]).reshape(1, 1, h)
    wd = wsp[6:7, :].reshape(1, 1, h)
    bsp3 = bsp_ref[...].reshape(1, 1, h)
    enc3 = jnp.maximum(
        cx3 * wc_x + cy3 * wc_y + nx3 * wn_x + ny3 * wn_y + dist3 * wd
        + bsp3, 0.0)
    nf3 = nf_ref[0][:, :h].reshape(t, K, h)

    watt = watt_ref[...]
    enc2 = enc3.reshape(t * K, h)
    nf2 = nf3.reshape(t * K, h)
    logits = (jnp.dot(enc2, watt[:h, :], preferred_element_type=jnp.float32)
              + jnp.dot(nf2, watt[h:, :], preferred_element_type=jnp.float32))
    l3 = logits.reshape(t, K, u)
    mx = jnp.max(l3, axis=1, keepdims=True)
    e = jnp.exp(l3 - mx)
    s = jnp.sum(e, axis=1, keepdims=True)
    scores = e / s
    agg_e = jnp.sum(scores[:, :, :h] * enc3, axis=1)
    agg_n = jnp.sum(scores[:, :, h:] * nf3, axis=1)
    wp = wp_ref[...]
    out = (jnp.dot(agg_e, wp[:h, :], preferred_element_type=jnp.float32)
           + jnp.dot(agg_n, wp[h:, :], preferred_element_type=jnp.float32)
           + bp_ref[...])
    f2 = jnp.maximum(out, 0.0)
    if epilogue is not None:
        f2 = epilogue(f2)
    out_ref[...] = f2.reshape(out_ref.shape).astype(out_ref.dtype)


def _nf_spec(rows, pack):
    w = PAD // pack
    if pack == 4:
        return pl.BlockSpec((1, rows, w), lambda b, i: (b, i, b))
    if pack == 2:
        return pl.BlockSpec((1, rows, w), lambda b, i: (b, i, b // 2))
    return pl.BlockSpec((1, rows, w), lambda b, i: (b, i, 0))


def _la_call(pc, npc, nf, wsp8, bsp, watt, wp_pad, bp_pad, od,
             pack, extras=None):
    b_dim, n, _ = pc.shape
    h = watt.shape[0] // 2
    rows = LA_T * K
    grid = (b_dim, n // LA_T)
    nb = n // LA_T
    in_specs = [
        pl.BlockSpec((1, LA_T, 2), lambda b, i: (b, i, 0)),
        pl.BlockSpec((1, rows, PAD), lambda b, i: (b, i, 0)),
        _nf_spec(rows, pack),
        pl.BlockSpec((8, h), lambda b, i: (0, 0)),
        pl.BlockSpec((1, h), lambda b, i: (0, 0)),
        pl.BlockSpec((2 * h, 2 * h), lambda b, i: (0, 0)),
        pl.BlockSpec((2 * h, od), lambda b, i: (0, 0)),
        pl.BlockSpec((1, od), lambda b, i: (0, 0)),
    ]
    args = [pc, npc, nf, wsp8, bsp, watt, wp_pad, bp_pad]
    if extras is None:
        kfn = functools.partial(_la_kernel, h=h, od=od)
        if pack == 4:
            out_spec = pl.BlockSpec((LA_T, od), lambda b, i: (i, b))
        elif pack == 2:
            out_spec = pl.BlockSpec(
                (LA_T, od), lambda b, i: ((b % 2) * nb + i, b // 2))
        else:
            out_spec = pl.BlockSpec(
                (LA_T, od), lambda b, i: (b * nb + i, 0))
        out_shape = jax.ShapeDtypeStruct((b_dim * n // pack, PAD),
                                         jnp.float32)
    else:
        fin, wm2, bm2, wsc, bsc = extras
        iu = wsc.shape[0]
        u = wm2.shape[0]
        in_specs += [
            pl.BlockSpec((1, LA_T, iu), lambda b, i: (b, i, 0)),
            pl.BlockSpec((u, u), lambda b, i: (0, 0)),
            pl.BlockSpec((1, u), lambda b, i: (0, 0)),
            pl.BlockSpec((iu, u), lambda b, i: (0, 0)),
            pl.BlockSpec((1, u), lambda b, i: (0, 0)),
        ]
        args += [fin, wm2, bm2, wsc, bsc]
        kfn = functools.partial(_la2_kernel, h=h, od=od)
        out_spec = pl.BlockSpec((1, LA_T, od), lambda b, i: (b, i, 0))
        out_shape = jax.ShapeDtypeStruct((b_dim, n, od), jnp.float32)
    return pl.pallas_call(
        kfn,
        grid=grid,
        in_specs=in_specs,
        out_specs=out_spec,
        out_shape=out_shape,
        interpret=INTERPRET,
    )(*args)


def _pad_w(w, width=PAD):
    out_w = w.shape[1]
    if out_w == width:
        return w
    return jnp.concatenate(
        [w, jnp.zeros((w.shape[0], width - out_w), w.dtype)], axis=1)


def _pad_b(b, width=PAD):
    if b.shape[0] == width:
        return b[None, :]
    return jnp.concatenate(
        [b, jnp.zeros((width - b.shape[0],), b.dtype)])[None, :]


def _pad8_rows(w):
    return jnp.concatenate([w, jnp.zeros((1, w.shape[1]), w.dtype)], axis=0)


def kernel(pc, feats, fc_kernel, fc_bias, drb_params):
    b_dim, n, _ = pc.shape

    f = _conv(feats, fc_kernel, fc_bias)

    pct = jnp.transpose(pc, (0, 2, 1))
    idx_g = _knn(pc, pct)
    idx_flat = idx_g.reshape(-1)
    idx_by_pack = {1: idx_flat,
                   2: idx_flat & (2 * n - 1),
                   4: idx_flat & (n - 1)}
    pc2d = pc.reshape(b_dim * n, DIMS)

    pc128 = _pad_w(pc2d)
    npc = _sc_gather128(pc128, idx_flat).reshape(b_dim, n * K, PAD)

    for p in drb_params:
        f_in = f
        iu = p['Wm1'].shape[0]
        h = p['Wm1'].shape[1]
        u = 2 * h
        pack = 1
        w = PAD // pack
        idx_p = idx_by_pack[pack]
        fmt = _fm(f_in.reshape(b_dim * n, iu),
                  _pad_w(p['Wm1'], w), _pad_b(p['bm1'], w), pack)
        nf = _sc_gather128(fmt, idx_p).reshape(b_dim, n * K, PAD)
        f1 = _la_call(pc, npc, nf, _pad8_rows(p['Wsp0']), p['bsp0'][None, :],
                      p['Watt0'], _pad_w(p['Wp0'], w), _pad_b(p['bp0'], w),
                      w, pack)
        nf2 = _sc_gather128(f1, idx_p).reshape(b_dim, n * K, PAD)
        f = _la_call(pc, npc, nf2, _pad8_rows(p['Wsp1']), p['bsp1'][None, :],
                     p['Watt1'], p['Wp1'], p['bp1'][None, :], u, pack,
                     extras=(f_in, p['Wm2'], p['bm2'][None, :],
                             p['Wsc'], p['bsc'][None, :]))
    return f
